# baseline (speedup 1.0000x reference)
import functools

import jax
import jax.numpy as jnp
from jax import lax
from jax.experimental import pallas as pl
from jax.experimental.pallas import tpu as pltpu
from jax.experimental.pallas import tpu_sc as plsc

H = 8
DH = 16
NC = 2
NS = 16
L = 16
HC = H // NC


def _qkv_body(x_ref, w3_ref, b3_ref, q_ref, kv_ref):
    x = x_ref[...]
    qkv = jnp.dot(x, w3_ref[...], preferred_element_type=jnp.float32) + b3_ref[...]
    scale = 1.0 / jnp.sqrt(jnp.float32(DH))
    q = qkv[:, :128] * scale
    k = qkv[:, 128:256]
    v = qkv[:, 256:]
    q_ref[0] = q[:, :64]
    q_ref[1] = q[:, 64:]
    kv_ref[0] = jnp.concatenate([k[:, :64], v[:, :64]], axis=1)
    kv_ref[1] = jnp.concatenate([k[:, 64:], v[:, 64:]], axis=1)


def _qkv_call(x, w3, b3, n_blk, blk):
    n = x.shape[0]
    return pl.pallas_call(
        _qkv_body,
        grid=(n_blk,),
        in_specs=[
            pl.BlockSpec((blk, 128), lambda i: (i, 0)),
            pl.BlockSpec((128, 384), lambda i: (0, 0)),
            pl.BlockSpec((1, 384), lambda i: (0, 0)),
        ],
        out_specs=[
            pl.BlockSpec((2, blk, 64), lambda i: (0, i, 0)),
            pl.BlockSpec((2, blk, 128), lambda i: (0, i, 0)),
        ],
        out_shape=[
            jax.ShapeDtypeStruct((2, n, 64), jnp.float32),
            jax.ShapeDtypeStruct((2, n, 128), jnp.float32),
        ],
    )(x, w3, b3)


def _make_edge_kernel(n, e):
    ep = e // NS
    c = 80
    nchunk = ep // c
    nrch = n // c

    mesh = plsc.VectorSubcoreMesh(core_axis_name="c", subcore_axis_name="s")

    @functools.partial(
        pl.kernel,
        out_type=[
            jax.ShapeDtypeStruct((NC, n, 8), jnp.float32),
            jax.ShapeDtypeStruct((NC, n, 64), jnp.float32),
        ],
        mesh=mesh,
        compiler_params=pltpu.CompilerParams(
            needs_layout_passes=False, use_tc_tiling_on_sc=False,
            disable_bounds_checks=True),
        scratch_types=[
            pltpu.VMEM((nchunk, c), jnp.int32),
            pltpu.VMEM((4, c), jnp.int32),
            pltpu.VMEM((c,), jnp.int32),
            pltpu.VMEM((c,), jnp.int32),
            pltpu.VMEM((c,), jnp.int32),
            pltpu.VMEM((c,), jnp.int32),
            pltpu.VMEM((c, 64), jnp.float32),
            pltpu.VMEM((c, 64), jnp.float32),
            pltpu.VMEM((c, 128), jnp.float32),
            pltpu.VMEM((c, 128), jnp.float32),
            pltpu.VMEM((c, 8), jnp.float32),
            pltpu.VMEM((c, 8), jnp.float32),
            pltpu.VMEM((c, 64), jnp.float32),
            pltpu.VMEM((c, 64), jnp.float32),
            pltpu.VMEM((64, 17), jnp.float32),
            pltpu.VMEM((64, 17), jnp.float32),
            pltpu.VMEM((L, 17), jnp.float32),
            pltpu.VMEM((L, 17), jnp.float32),
            pltpu.VMEM_SHARED((n, 8), jnp.float32),
            pltpu.VMEM_SHARED((n, 64), jnp.float32),
            pltpu.SemaphoreType.DMA,
            pltpu.SemaphoreType.DMA,
            pltpu.SemaphoreType.DMA,
            pltpu.SemaphoreType.DMA,
            pltpu.SemaphoreType.DMA,
            pltpu.SemaphoreType.DMA,
            pltpu.SemaphoreType.DMA,
            pltpu.SemaphoreType.DMA,
        ],
    )
    def edge_kernel(pk_hbm, q_hbm, kv_hbm, den_out, wv_out,
                    pkv, tsc2, tq0, tq1, sq0, sq1, qr0, qr1, kvr0, kvr1,
                    exb0, exb1, wvb0, wvb1, prod_ta, prod_tb, exwa, exwb, den_sh, wv_sh,
                    sem_q0, sem_q1, sem_kv0, sem_kv1,
                    sem_d0, sem_d1, sem_w0, sem_w1):
        cid = lax.axis_index("c")
        sid = lax.axis_index("s")
        cid_n = jnp.full((L,), cid * n, jnp.int32)

        tq = [tq0, tq1]
        sq = [sq0, sq1]
        qr = [qr0, qr1]
        kvr = [kvr0, kvr1]
        exb = [exb0, exb1]
        wvb = [wvb0, wvb1]
        sem_q = [sem_q0, sem_q1]
        sem_kv = [sem_kv0, sem_kv1]
        sem_d = [sem_d0, sem_d1]
        sem_w = [sem_w0, sem_w1]

        zero16 = jnp.zeros((L,), jnp.float32)
        iota = lax.iota(jnp.int32, L)

        pltpu.sync_copy(pk_hbm.at[sid], pkv)

        def _zw(i, _):
            for jj in range(4):
                wvb0[i, pl.ds(jj * L, L)] = zero16
            return 0
        lax.fori_loop(0, c, _zw, 0)
        def _ze(i, _):
            flat = i * L + iota
            rows = lax.shift_right_logical(flat, 3)
            cols = lax.bitwise_and(flat, jnp.full((L,), 7, jnp.int32))
            plsc.store_scatter(exb0, [rows, cols], zero16)
            plsc.store_scatter(exb1, [rows, cols], zero16)
            return 0
        lax.fori_loop(0, c * 8 // L, _ze, 0)
        for jj in range(8):
            blkid = sid + jj * NS
            @pl.when(blkid < nrch)
            def _():
                pltpu.sync_copy(exb0, den_sh.at[pl.ds(blkid * c, c)])
                pltpu.sync_copy(wvb0, wv_sh.at[pl.ds(blkid * c, c)])
        plsc.subcore_barrier()

        lo16 = jnp.full((L,), 0xFFFF, jnp.int32)

        def _prep_idx(i, b):
            for jj in range(c // L):
                sl = pl.ds(jj * L, L)
                val = pkv[i, sl]
                t = lax.bitwise_and(val, lo16)
                tq[b][sl] = t + cid_n
                sq[b][sl] = lax.shift_right_logical(val, 16) + cid_n

        def _issue(i, b):
            _prep_idx(i, b)
            pltpu.async_copy(q_hbm.at[tq[b]], qr[b], sem_q[b])
            pltpu.async_copy(kv_hbm.at[sq[b]], kvr[b], sem_kv[b])

        for b in range(2):
            _issue(jnp.int32(b), b)

        def chunk_pair_body(t, _):
            for b in range(2):
                i = t * 2 + b
                pltpu.make_async_copy(q_hbm.at[tq[b]], qr[b], sem_q[b]).wait()
                pltpu.make_async_copy(kv_hbm.at[sq[b]], kvr[b], sem_kv[b]).wait()

                @pl.when(i >= 2)
                def _(b=b):
                    pltpu.make_async_copy(
                        exb[b], den_sh.at[tsc2.at[b]], sem_d[b]).wait()
                    pltpu.make_async_copy(
                        wvb[b], wv_sh.at[tsc2.at[b]], sem_w[b]).wait()

                si4 = lax.bitwise_and(i, 3)
                for jj in range(c // L):
                    sl = pl.ds(jj * L, L)
                    tsc2[si4, sl] = tq[b][sl] - cid_n

                for g in range(c // L):
                    prod_t = prod_ta if g % 2 == 0 else prod_tb
                    exw = exwa if g % 2 == 0 else exwb
                    el = iota + g * L
                    for e16 in range(L):
                        e_loc = g * L + e16
                        e_vec = jnp.full((L,), e16, jnp.int32)
                        for j4 in range(4):
                            qv = qr[b][e_loc, pl.ds(j4 * DH, DH)]
                            kv_ = kvr[b][e_loc, pl.ds(j4 * DH, DH)]
                            plsc.store_scatter(
                                prod_t, [iota + j4 * DH, e_vec], qv * kv_)
                    for h in range(HC):
                        acc = None
                        for d in range(DH):
                            r = prod_t[h * DH + d, pl.ds(0, L)]
                            acc = r if acc is None else acc + r
                        ex = jnp.exp(acc)
                        hv = jnp.full((L,), h, jnp.int32)
                        plsc.store_scatter(exb[b], [el, hv], ex)
                        plsc.store_scatter(exw, [iota, jnp.full((L,), h, jnp.int32)], ex)
                    for e16 in range(L):
                        e_loc = g * L + e16
                        ex_row = exw[e16, pl.ds(0, L)]
                        for h in range(HC):
                            vv = kvr[b][e_loc, pl.ds(64 + h * DH, DH)]
                            wvb[b][e_loc, pl.ds(h * DH, DH)] = vv * ex_row[h]

                pltpu.async_copy(exb[b], den_sh.at[tsc2.at[si4]], sem_d[b],
                                 add=True)
                pltpu.async_copy(wvb[b], wv_sh.at[tsc2.at[si4]], sem_w[b],
                                 add=True)

                nxt = i + 2
                @pl.when(nxt < nchunk)
                def _(b=b, nxt=nxt):
                    _issue(nxt, b)
            return 0
        lax.fori_loop(0, nchunk // 2, chunk_pair_body, 0)

        for b in range(2):
            pltpu.make_async_copy(
                exb[b], den_sh.at[tsc2.at[b]], sem_d[b]).wait()
            pltpu.make_async_copy(
                wvb[b], wv_sh.at[tsc2.at[b]], sem_w[b]).wait()
        plsc.subcore_barrier()

        for jj in range(8):
            blkid = sid + jj * NS
            @pl.when(blkid < nrch)
            def _():
                sl = pl.ds(blkid * c, c)
                pltpu.sync_copy(den_sh.at[sl], den_out.at[cid, sl])
                pltpu.sync_copy(wv_sh.at[sl], wv_out.at[cid, sl])

    return edge_kernel


def _finish_body(den_ref, wv_ref, x_ref, wo_ref, bo_ref, g_ref, b_ref, o_ref):
    hh = lax.broadcasted_iota(jnp.int32, (8, 128), 0)
    jj = lax.broadcasted_iota(jnp.int32, (8, 128), 1)
    e0 = ((jj // DH == hh) & (hh < HC)).astype(jnp.float32)
    e1 = ((jj // DH == hh + HC) & (hh < HC)).astype(jnp.float32)
    den_full = (jnp.dot(den_ref[0], e0, preferred_element_type=jnp.float32)
                + jnp.dot(den_ref[1], e1, preferred_element_type=jnp.float32))
    wv = jnp.concatenate([wv_ref[0], wv_ref[1]], axis=-1)
    nodes = jnp.where(den_full > 0.0, wv / den_full, 0.0)
    o = jnp.dot(nodes, wo_ref[...], preferred_element_type=jnp.float32) + bo_ref[...]
    res = o + x_ref[...]
    mu = jnp.mean(res, axis=-1, keepdims=True)
    var = jnp.mean((res - mu) ** 2, axis=-1, keepdims=True)
    normed = (res - mu) * lax.rsqrt(var + 1e-5)
    o_ref[...] = normed * g_ref[...] + b_ref[...]


def _finish_call(den_p, wv_p, x, wo_t, bo, gamma, beta, n_blk, blk):
    return pl.pallas_call(
        _finish_body,
        grid=(n_blk,),
        in_specs=[
            pl.BlockSpec((2, blk, 8), lambda i: (0, i, 0)),
            pl.BlockSpec((2, blk, 64), lambda i: (0, i, 0)),
            pl.BlockSpec((blk, 128), lambda i: (i, 0)),
            pl.BlockSpec((128, 128), lambda i: (0, 0)),
            pl.BlockSpec((1, 128), lambda i: (0, 0)),
            pl.BlockSpec((1, 128), lambda i: (0, 0)),
            pl.BlockSpec((1, 128), lambda i: (0, 0)),
        ],
        out_specs=pl.BlockSpec((blk, 128), lambda i: (i, 0)),
        out_shape=jax.ShapeDtypeStruct((x.shape[0], 128), jnp.float32),
    )(den_p, wv_p, x, wo_t, bo, gamma, beta)


def kernel(node_features, edge_index, Wq, bq, Wk, bk, Wv, bv, Wo, bo, gamma, beta):
    b, n, d = node_features.shape
    e = edge_index.shape[-1]
    x = node_features.reshape(n, d)

    w3 = jnp.concatenate([Wq.T, Wk.T, Wv.T], axis=1)
    b3 = jnp.concatenate([bq, bk, bv]).reshape(1, 384)

    blk = 1000
    n_blk = n // blk
    q, kv = _qkv_call(x, w3, b3, n_blk, blk)
    q_tbl = q.reshape(2 * n, 64)
    kv_tbl = kv.reshape(2 * n, 128)

    ep = e // NS
    c = 80
    es = edge_index.reshape(2, e)
    packed = (es[1] | (es[0] << 16)).reshape(NS, ep // c, c)

    den_p, wv_p = _make_edge_kernel(n, e)(packed, q_tbl, kv_tbl)

    out = _finish_call(den_p, wv_p, x, Wo.T, bo.reshape(1, 128),
                       gamma.reshape(1, 128), beta.reshape(1, 128), n_blk, blk)
    return out.reshape(b, n, d)


# speedup vs baseline: 1.8948x; 1.8948x over previous
import functools

import jax
import jax.numpy as jnp
from jax import lax
from jax.experimental import pallas as pl
from jax.experimental.pallas import tpu as pltpu
from jax.experimental.pallas import tpu_sc as plsc

H = 8
DH = 16
NC = 2
NS = 16
L = 16
HC = H // NC


def _qkv_body(x_ref, w3_ref, b3_ref, q_ref, kv_ref):
    x = x_ref[...]
    qkv = jnp.dot(x, w3_ref[...], preferred_element_type=jnp.float32) + b3_ref[...]
    scale = 1.0 / jnp.sqrt(jnp.float32(DH))
    q = qkv[:, :128] * scale
    k = qkv[:, 128:256]
    v = qkv[:, 256:]
    q_ref[0] = q[:, :64]
    q_ref[1] = q[:, 64:]
    kv_ref[0] = jnp.concatenate([k[:, :64], v[:, :64]], axis=1)
    kv_ref[1] = jnp.concatenate([k[:, 64:], v[:, 64:]], axis=1)


def _qkv_call(x, w3, b3, n_blk, blk):
    n = x.shape[0]
    return pl.pallas_call(
        _qkv_body,
        grid=(n_blk,),
        in_specs=[
            pl.BlockSpec((blk, 128), lambda i: (i, 0)),
            pl.BlockSpec((128, 384), lambda i: (0, 0)),
            pl.BlockSpec((1, 384), lambda i: (0, 0)),
        ],
        out_specs=[
            pl.BlockSpec((2, blk, 64), lambda i: (0, i, 0)),
            pl.BlockSpec((2, blk, 128), lambda i: (0, i, 0)),
        ],
        out_shape=[
            jax.ShapeDtypeStruct((2, n, 64), jnp.float32),
            jax.ShapeDtypeStruct((2, n, 128), jnp.float32),
        ],
    )(x, w3, b3)


def _make_edge_kernel(n, e):
    ep = e // NS
    c = 80
    nchunk = ep // c
    nrch = n // c

    mesh = plsc.VectorSubcoreMesh(core_axis_name="c", subcore_axis_name="s")

    @functools.partial(
        pl.kernel,
        out_type=[
            jax.ShapeDtypeStruct((NC, n, 8), jnp.float32),
            jax.ShapeDtypeStruct((NC, n, 64), jnp.float32),
        ],
        mesh=mesh,
        compiler_params=pltpu.CompilerParams(
            needs_layout_passes=False, use_tc_tiling_on_sc=False,
            disable_bounds_checks=True),
        scratch_types=[
            pltpu.VMEM((nchunk, c), jnp.int32),
            pltpu.VMEM((nchunk, c), jnp.int32),
            pltpu.VMEM((c,), jnp.int32),
            pltpu.VMEM((c,), jnp.int32),
            pltpu.VMEM((c,), jnp.int32),
            pltpu.VMEM((c,), jnp.int32),
            pltpu.VMEM((c, 64), jnp.float32),
            pltpu.VMEM((c, 64), jnp.float32),
            pltpu.VMEM((c, 128), jnp.float32),
            pltpu.VMEM((c, 128), jnp.float32),
            pltpu.VMEM((c, 8), jnp.float32),
            pltpu.VMEM((c, 8), jnp.float32),
            pltpu.VMEM((c, 64), jnp.float32),
            pltpu.VMEM((c, 64), jnp.float32),
            pltpu.VMEM((64, 17), jnp.float32),
            pltpu.VMEM((L, 17), jnp.float32),
            pltpu.VMEM_SHARED((n, 8), jnp.float32),
            pltpu.VMEM_SHARED((n, 64), jnp.float32),
            pltpu.SemaphoreType.DMA,
            pltpu.SemaphoreType.DMA,
            pltpu.SemaphoreType.DMA,
            pltpu.SemaphoreType.DMA,
            pltpu.SemaphoreType.DMA,
            pltpu.SemaphoreType.DMA,
            pltpu.SemaphoreType.DMA,
            pltpu.SemaphoreType.DMA,
        ],
    )
    def edge_kernel(tgt_hbm, src_hbm, q_hbm, kv_hbm, den_out, wv_out,
                    tgtv, srcv, tq0, tq1, sq0, sq1, qr0, qr1, kvr0, kvr1,
                    exb0, exb1, wvb0, wvb1, prod_t, exw, den_sh, wv_sh,
                    sem_q0, sem_q1, sem_kv0, sem_kv1,
                    sem_d0, sem_d1, sem_w0, sem_w1):
        cid = lax.axis_index("c")
        sid = lax.axis_index("s")
        cid_n = jnp.full((L,), cid * n, jnp.int32)

        tq = [tq0, tq1]
        sq = [sq0, sq1]
        qr = [qr0, qr1]
        kvr = [kvr0, kvr1]
        exb = [exb0, exb1]
        wvb = [wvb0, wvb1]
        sem_q = [sem_q0, sem_q1]
        sem_kv = [sem_kv0, sem_kv1]
        sem_d = [sem_d0, sem_d1]
        sem_w = [sem_w0, sem_w1]

        zero16 = jnp.zeros((L,), jnp.float32)
        iota = lax.iota(jnp.int32, L)

        pltpu.sync_copy(tgt_hbm.at[sid], tgtv)
        pltpu.sync_copy(src_hbm.at[sid], srcv)

        def _zw(i, _):
            for jj in range(4):
                wvb0[i, pl.ds(jj * L, L)] = zero16
            return 0
        lax.fori_loop(0, c, _zw, 0)
        def _ze(i, _):
            flat = i * L + iota
            rows = lax.shift_right_logical(flat, 3)
            cols = lax.bitwise_and(flat, jnp.full((L,), 7, jnp.int32))
            plsc.store_scatter(exb0, [rows, cols], zero16)
            plsc.store_scatter(exb1, [rows, cols], zero16)
            return 0
        lax.fori_loop(0, c * 8 // L, _ze, 0)
        for jj in range(8):
            blkid = sid + jj * NS
            @pl.when(blkid < nrch)
            def _():
                pltpu.sync_copy(exb0, den_sh.at[pl.ds(blkid * c, c)])
                pltpu.sync_copy(wvb0, wv_sh.at[pl.ds(blkid * c, c)])
        plsc.subcore_barrier()

        def _prep_idx(i, b):
            for jj in range(c // L):
                sl = pl.ds(jj * L, L)
                tq[b][sl] = tgtv[i, sl] + cid_n
                sq[b][sl] = srcv[i, sl] + cid_n

        def _issue(i, b):
            _prep_idx(i, b)
            pltpu.async_copy(q_hbm.at[tq[b]], qr[b], sem_q[b])
            pltpu.async_copy(kv_hbm.at[sq[b]], kvr[b], sem_kv[b])

        for b in range(2):
            _issue(jnp.int32(b), b)

        def chunk_pair_body(t, _):
            for b in range(2):
                i = t * 2 + b
                pltpu.make_async_copy(q_hbm.at[tq[b]], qr[b], sem_q[b]).wait()
                pltpu.make_async_copy(kv_hbm.at[sq[b]], kvr[b], sem_kv[b]).wait()

                @pl.when(i >= 2)
                def _(b=b):
                    pltpu.make_async_copy(
                        exb[b], den_sh.at[tgtv.at[0]], sem_d[b]).wait()
                    pltpu.make_async_copy(
                        wvb[b], wv_sh.at[tgtv.at[0]], sem_w[b]).wait()

                def group_body(g, _, b=b):
                    el = iota + g * L
                    for e2 in range(L // 2):
                        locs = [g * L + e2 * 2, g * L + e2 * 2 + 1]
                        vecs = [jnp.full((L,), e2 * 2, jnp.int32),
                                jnp.full((L,), e2 * 2 + 1, jnp.int32)]
                        qs = [qr[b][e_loc, pl.ds(j * DH, DH)]
                              for e_loc in locs for j in range(4)]
                        ks = [kvr[b][e_loc, pl.ds(j * DH, DH)]
                              for e_loc in locs for j in range(4)]
                        ps = [q * k for q, k in zip(qs, ks)]
                        for ee in range(2):
                            for j in range(4):
                                plsc.store_scatter(
                                    prod_t, [iota + j * DH, vecs[ee]],
                                    ps[ee * 4 + j])
                    for h in range(HC):
                        rs = [prod_t[h * DH + d, pl.ds(0, L)]
                              for d in range(DH)]
                        while len(rs) > 1:
                            rs = [rs[i] + rs[i + 1]
                                  for i in range(0, len(rs), 2)]
                        ex = jnp.exp(rs[0])
                        hv = jnp.full((L,), h, jnp.int32)
                        plsc.store_scatter(exb[b], [el, hv], ex)
                        plsc.store_scatter(exw, [iota, hv], ex)
                    for e2 in range(L // 2):
                        locs = [g * L + e2 * 2, g * L + e2 * 2 + 1]
                        rows = [exw[e2 * 2 + ee, pl.ds(0, L)]
                                for ee in range(2)]
                        vvs = [kvr[b][e_loc, pl.ds(64 + h * DH, DH)]
                               for e_loc in locs for h in range(HC)]
                        outs = [vvs[ee * 4 + h] * rows[ee][h]
                                for ee in range(2) for h in range(HC)]
                        for ee in range(2):
                            for h in range(HC):
                                wvb[b][locs[ee], pl.ds(h * DH, DH)] = (
                                    outs[ee * 4 + h])
                    return 0
                lax.fori_loop(0, c // L, group_body, 0)

                pltpu.async_copy(exb[b], den_sh.at[tgtv.at[i]], sem_d[b],
                                 add=True)
                pltpu.async_copy(wvb[b], wv_sh.at[tgtv.at[i]], sem_w[b],
                                 add=True)

                nxt = i + 2
                @pl.when(nxt < nchunk)
                def _(b=b, nxt=nxt):
                    _issue(nxt, b)
            return 0
        lax.fori_loop(0, nchunk // 2, chunk_pair_body, 0)

        for b in range(2):
            pltpu.make_async_copy(
                exb[b], den_sh.at[tgtv.at[0]], sem_d[b]).wait()
            pltpu.make_async_copy(
                wvb[b], wv_sh.at[tgtv.at[0]], sem_w[b]).wait()
        plsc.subcore_barrier()

        for jj in range(8):
            blkid = sid + jj * NS
            @pl.when(blkid < nrch)
            def _():
                sl = pl.ds(blkid * c, c)
                pltpu.sync_copy(den_sh.at[sl], den_out.at[cid, sl])
                pltpu.sync_copy(wv_sh.at[sl], wv_out.at[cid, sl])

    return edge_kernel


def _finish_body(den_ref, wv_ref, x_ref, wo_ref, bo_ref, g_ref, b_ref, o_ref):
    hh = lax.broadcasted_iota(jnp.int32, (8, 128), 0)
    jj = lax.broadcasted_iota(jnp.int32, (8, 128), 1)
    e0 = ((jj // DH == hh) & (hh < HC)).astype(jnp.float32)
    e1 = ((jj // DH == hh + HC) & (hh < HC)).astype(jnp.float32)
    den_full = (jnp.dot(den_ref[0], e0, preferred_element_type=jnp.float32)
                + jnp.dot(den_ref[1], e1, preferred_element_type=jnp.float32))
    wv = jnp.concatenate([wv_ref[0], wv_ref[1]], axis=-1)
    nodes = jnp.where(den_full > 0.0, wv / den_full, 0.0)
    o = jnp.dot(nodes, wo_ref[...], preferred_element_type=jnp.float32) + bo_ref[...]
    res = o + x_ref[...]
    mu = jnp.mean(res, axis=-1, keepdims=True)
    var = jnp.mean((res - mu) ** 2, axis=-1, keepdims=True)
    normed = (res - mu) * lax.rsqrt(var + 1e-5)
    o_ref[...] = normed * g_ref[...] + b_ref[...]


def _finish_call(den_p, wv_p, x, wo_t, bo, gamma, beta, n_blk, blk):
    return pl.pallas_call(
        _finish_body,
        grid=(n_blk,),
        in_specs=[
            pl.BlockSpec((2, blk, 8), lambda i: (0, i, 0)),
            pl.BlockSpec((2, blk, 64), lambda i: (0, i, 0)),
            pl.BlockSpec((blk, 128), lambda i: (i, 0)),
            pl.BlockSpec((128, 128), lambda i: (0, 0)),
            pl.BlockSpec((1, 128), lambda i: (0, 0)),
            pl.BlockSpec((1, 128), lambda i: (0, 0)),
            pl.BlockSpec((1, 128), lambda i: (0, 0)),
        ],
        out_specs=pl.BlockSpec((blk, 128), lambda i: (i, 0)),
        out_shape=jax.ShapeDtypeStruct((x.shape[0], 128), jnp.float32),
    )(den_p, wv_p, x, wo_t, bo, gamma, beta)


def kernel(node_features, edge_index, Wq, bq, Wk, bk, Wv, bv, Wo, bo, gamma, beta):
    b, n, d = node_features.shape
    e = edge_index.shape[-1]
    x = node_features.reshape(n, d)

    w3 = jnp.concatenate([Wq.T, Wk.T, Wv.T], axis=1)
    b3 = jnp.concatenate([bq, bk, bv]).reshape(1, 384)

    blk = 1000
    n_blk = n // blk
    q, kv = _qkv_call(x, w3, b3, n_blk, blk)
    q_tbl = q.reshape(2 * n, 64)
    kv_tbl = kv.reshape(2 * n, 128)

    ep = e // NS
    c = 80
    src = edge_index.reshape(2, e)[0].reshape(NS, ep // c, c)
    tgt = edge_index.reshape(2, e)[1].reshape(NS, ep // c, c)

    den_p, wv_p = _make_edge_kernel(n, e)(tgt, src, q_tbl, kv_tbl)

    out = _finish_call(den_p, wv_p, x, Wo.T, bo.reshape(1, 128),
                       gamma.reshape(1, 128), beta.reshape(1, 128), n_blk, blk)
    return out.reshape(b, n, d)
